# baseline (speedup 1.0000x reference)
import functools

import jax
import jax.numpy as jnp
from jax import lax
from jax.experimental import pallas as pl
from jax.experimental.pallas import tpu as pltpu
from jax.experimental.pallas import tpu_sc as plsc


def _sort_body(s_ref, idx_ref, fidx_ref):
    B, N = s_ref.shape
    K = N // 2
    s = s_ref[...]
    idx = lax.broadcasted_iota(jnp.int32, (B, N), 1)
    pos = lax.broadcasted_iota(jnp.int32, (B, N), 1)

    k = 2
    while k <= N:
        j = k // 2
        while j >= 1:
            s_m = pltpu.roll(s, j, 1)
            s_p = pltpu.roll(s, N - j, 1)
            i_m = pltpu.roll(idx, j, 1)
            i_p = pltpu.roll(idx, N - j, 1)
            hi = (pos & j) != 0
            s_part = jnp.where(hi, s_m, s_p)
            i_part = jnp.where(hi, i_m, i_p)
            first = (s > s_part) | ((s == s_part) & (idx < i_part))
            dirf = (pos & k) == 0
            want_first = jnp.logical_not(hi) == dirf
            take = jnp.logical_xor(first, want_first)
            s = jnp.where(take, s_part, s)
            idx = jnp.where(take, i_part, idx)
            j //= 2
        k *= 2

    top = idx[:, :K]
    idx_ref[...] = top
    off = lax.broadcasted_iota(jnp.int32, (B, K), 0) * N
    fidx_ref[...] = top + off


def _topk(scores):
    B, N = scores.shape
    K = N // 2
    return pl.pallas_call(
        _sort_body,
        out_shape=(
            jax.ShapeDtypeStruct((B, K), jnp.int32),
            jax.ShapeDtypeStruct((B, K), jnp.int32),
        ),
    )(scores)


_INFO = plsc.get_sparse_core_info()
_NC = _INFO.num_cores
_NS = _INFO.num_subcores
_NW = _NC * _NS


def _make_gather(R, D, CHUNK):
    rpw = R // _NW
    n_chunks = rpw // CHUNK
    mesh = plsc.VectorSubcoreMesh(core_axis_name="c", subcore_axis_name="s")

    @functools.partial(
        pl.kernel,
        mesh=mesh,
        out_type=jax.ShapeDtypeStruct((R, D), jnp.float32),
        scratch_types=[
            pltpu.VMEM((rpw,), jnp.int32),
            pltpu.VMEM((CHUNK, D), jnp.float32),
            pltpu.VMEM((CHUNK, D), jnp.float32),
            pltpu.SemaphoreType.DMA,
            pltpu.SemaphoreType.DMA,
        ],
    )
    def gather(x_hbm, fidx_hbm, out_hbm, idx_v, buf0, buf1, sem0, sem1):
        wid = lax.axis_index("s") * _NC + lax.axis_index("c")
        base = wid * rpw
        pltpu.sync_copy(fidx_hbm.at[pl.ds(base, rpw)], idx_v)

        bufs = (buf0, buf1)
        sems = (sem0, sem1)

        def start(g):
            return pltpu.async_copy(
                x_hbm.at[idx_v.at[pl.ds(g * CHUNK, CHUNK)]],
                bufs[g % 2], sems[g % 2])

        pend = start(0)
        for g in range(n_chunks):
            nxt = start(g + 1) if g + 1 < n_chunks else None
            pend.wait()
            pltpu.sync_copy(bufs[g % 2], out_hbm.at[pl.ds(base + g * CHUNK, CHUNK)])
            pend = nxt

    return gather


def kernel(x, W1, b1, W2, b2):
    B, N, D = x.shape
    K = N // 2
    h = jax.nn.gelu(jnp.einsum('bnd,dh->bnh', x, W1) + b1, approximate=False)
    scores = (jnp.einsum('bnh,ho->bno', h, W2) + b2)[..., 0]
    idx, fidx = _topk(scores)
    xf = x.reshape(B * N, D)
    out = _make_gather(B * K, D, 32)(xf, fidx.reshape(B * K))
    return out.reshape(B, K, D), idx
